# baseline (speedup 1.0000x reference)
import functools

import jax
import jax.numpy as jnp
from jax import lax
from jax.experimental import pallas as pl
from jax.experimental.pallas import tpu as pltpu
from jax.experimental.pallas import tpu_sc as plsc

N_NODES = 10000
N_EDGES = 160000
IN_FEATS = 256
HIDDEN = 256
NUM_CLASSES = 64

NC = 2
NS = 16
NBUF = 3

CHUNK1 = 80
EPT1 = N_EDGES // NS
NCHUNK1 = EPT1 // CHUNK1

CHUNK2 = 40
EPT2 = N_EDGES // (NC * NS)
NCHUNK2 = EPT2 // CHUNK2

ZB = 632
ZLAST = N_NODES - (NS - 1) * ZB


def _linear1(x, w, b):
    blk = 1000
    half = HIDDEN // 2

    def body(x_ref, w_ref, b_ref, lo_ref, hi_ref):
        h = jnp.dot(x_ref[...], w_ref[...], preferred_element_type=jnp.float32)
        h = h + b_ref[...]
        lo_ref[...] = h[:, :half]
        hi_ref[...] = h[:, half:]

    return pl.pallas_call(
        body,
        grid=(N_NODES // blk,),
        in_specs=[
            pl.BlockSpec((blk, IN_FEATS), lambda i: (i, 0)),
            pl.BlockSpec((IN_FEATS, HIDDEN), lambda i: (0, 0)),
            pl.BlockSpec((1, HIDDEN), lambda i: (0, 0)),
        ],
        out_specs=[
            pl.BlockSpec((blk, half), lambda i: (i, 0)),
            pl.BlockSpec((blk, half), lambda i: (i, 0)),
        ],
        out_shape=[jax.ShapeDtypeStruct((N_NODES, half), jnp.float32)] * 2,
    )(x, w, b)


def _linear2(lo, hi, wa, wb, b):
    blk = 1000

    def body(lo_ref, hi_ref, wa_ref, wb_ref, b_ref, o_ref):
        h = jnp.dot(jnp.maximum(lo_ref[...], 0.0), wa_ref[...],
                    preferred_element_type=jnp.float32)
        h = h + jnp.dot(jnp.maximum(hi_ref[...], 0.0), wb_ref[...],
                        preferred_element_type=jnp.float32)
        o_ref[...] = h + b_ref[...]

    return pl.pallas_call(
        body,
        grid=(N_NODES // blk,),
        in_specs=[
            pl.BlockSpec((blk, HIDDEN // 2), lambda i: (i, 0)),
            pl.BlockSpec((blk, HIDDEN // 2), lambda i: (i, 0)),
            pl.BlockSpec((HIDDEN // 2, NUM_CLASSES), lambda i: (0, 0)),
            pl.BlockSpec((HIDDEN // 2, NUM_CLASSES), lambda i: (0, 0)),
            pl.BlockSpec((1, NUM_CLASSES), lambda i: (0, 0)),
        ],
        out_specs=pl.BlockSpec((blk, NUM_CLASSES), lambda i: (i, 0)),
        out_shape=jax.ShapeDtypeStruct((N_NODES, NUM_CLASSES), jnp.float32),
    )(lo, hi, wa, wb, b)


def _final_add(a, b):
    blk = 1000

    def body(a_ref, b_ref, o_ref):
        o_ref[...] = a_ref[...] + b_ref[...]

    return pl.pallas_call(
        body,
        grid=(N_NODES // blk,),
        in_specs=[
            pl.BlockSpec((blk, NUM_CLASSES), lambda i: (i, 0)),
            pl.BlockSpec((blk, NUM_CLASSES), lambda i: (i, 0)),
        ],
        out_specs=pl.BlockSpec((blk, NUM_CLASSES), lambda i: (i, 0)),
        out_shape=jax.ShapeDtypeStruct((N_NODES, NUM_CLASSES), jnp.float32),
    )(a, b)


def _zero_fill(buf, nrows, d):
    per_row = d // 16

    def body(i, carry):
        r = i // per_row
        cc = lax.rem(i, per_row)
        buf[r, pl.ds(cc * 16, 16)] = jnp.zeros((16,), jnp.float32)
        return carry

    lax.fori_loop(0, nrows * per_row, body, 0)


def _zero_agg(zbuf, agg, r0, nrows, chunk):
    nfull, tail = nrows // chunk, nrows % chunk
    for kk in range(nfull):
        pltpu.sync_copy(zbuf, agg.at[pl.ds(r0 + kk * chunk, chunk)])
    if tail:
        pltpu.sync_copy(zbuf.at[pl.ds(0, tail)],
                        agg.at[pl.ds(r0 + nfull * chunk, tail)])


def _run_chunks(h_hbm, sidx, didx, rows, sems, agg, nchunk):
    def chunk_body(jj, carry):
        base = NBUF * jj
        gs = [pltpu.async_copy(h_hbm.at[sidx.at[base + b]], rows[b], sems[b])
              for b in range(NBUF)]
        for b in range(NBUF):
            gs[b].wait()
            pltpu.sync_copy(rows[b], agg.at[didx.at[base + b]], add=True)
        return carry

    lax.fori_loop(0, nchunk // NBUF, chunk_body, 0)
    for j in range(nchunk - nchunk % NBUF, nchunk):
        g = pltpu.async_copy(h_hbm.at[sidx.at[j]], rows[0], sems[0])
        g.wait()
        pltpu.sync_copy(rows[0], agg.at[didx.at[j]], add=True)


def _scatter1():
    d = HIDDEN // 2
    mesh = plsc.VectorSubcoreMesh(core_axis_name="c", subcore_axis_name="s")

    @functools.partial(
        pl.kernel,
        out_type=[jax.ShapeDtypeStruct((N_NODES, d), jnp.float32)] * 2,
        mesh=mesh,
        scratch_types=[
            pltpu.VMEM((NCHUNK1, CHUNK1), jnp.int32),
            pltpu.VMEM((NCHUNK1, CHUNK1), jnp.int32),
            [pltpu.VMEM((CHUNK1, d), jnp.float32)] * NBUF,
            pltpu.VMEM_SHARED((N_NODES, d), jnp.float32),
            [pltpu.SemaphoreType.DMA] * NBUF,
        ],
        compiler_params=pltpu.CompilerParams(use_tc_tiling_on_sc=False),
    )
    def k(h_lo, h_hi, src_hbm, dst_hbm, out_lo, out_hi,
          sidx, didx, rows, agg, sems):
        c = lax.axis_index("c")
        s = lax.axis_index("s")
        r0 = pl.multiple_of(s * ZB, 8)

        pltpu.sync_copy(src_hbm.at[s], sidx)
        pltpu.sync_copy(dst_hbm.at[s], didx)

        _zero_fill(rows[0], CHUNK1, d)

        @pl.when(s < NS - 1)
        def _():
            _zero_agg(rows[0], agg, r0, ZB, CHUNK1)

        @pl.when(s == NS - 1)
        def _():
            _zero_agg(rows[0], agg, r0, ZLAST, CHUNK1)

        plsc.subcore_barrier()

        @pl.when(c == 0)
        def _():
            _run_chunks(h_lo, sidx, didx, rows, sems, agg, NCHUNK1)

        @pl.when(c == 1)
        def _():
            _run_chunks(h_hi, sidx, didx, rows, sems, agg, NCHUNK1)

        plsc.subcore_barrier()

        def copy_out(out_hbm):
            @pl.when(s < NS - 1)
            def _():
                pltpu.sync_copy(agg.at[pl.ds(r0, ZB)], out_hbm.at[pl.ds(r0, ZB)])

            @pl.when(s == NS - 1)
            def _():
                pltpu.sync_copy(agg.at[pl.ds(r0, ZLAST)],
                                out_hbm.at[pl.ds(r0, ZLAST)])

        @pl.when(c == 0)
        def _():
            copy_out(out_lo)

        @pl.when(c == 1)
        def _():
            copy_out(out_hi)

    return k


def _scatter2():
    d = NUM_CLASSES
    mesh = plsc.VectorSubcoreMesh(core_axis_name="c", subcore_axis_name="s")

    @functools.partial(
        pl.kernel,
        out_type=[jax.ShapeDtypeStruct((N_NODES, d), jnp.float32)] * 2,
        mesh=mesh,
        scratch_types=[
            pltpu.VMEM((NCHUNK2, CHUNK2), jnp.int32),
            pltpu.VMEM((NCHUNK2, CHUNK2), jnp.int32),
            [pltpu.VMEM((CHUNK2, d), jnp.float32)] * NBUF,
            pltpu.VMEM_SHARED((N_NODES, d), jnp.float32),
            [pltpu.SemaphoreType.DMA] * NBUF,
        ],
        compiler_params=pltpu.CompilerParams(use_tc_tiling_on_sc=False),
    )
    def k(h2, src_hbm, dst_hbm, out_a, out_b,
          sidx, didx, rows, agg, sems):
        c = lax.axis_index("c")
        s = lax.axis_index("s")
        r0 = pl.multiple_of(s * ZB, 8)

        pltpu.sync_copy(src_hbm.at[c, s], sidx)
        pltpu.sync_copy(dst_hbm.at[c, s], didx)

        _zero_fill(rows[0], CHUNK2, d)

        @pl.when(s < NS - 1)
        def _():
            _zero_agg(rows[0], agg, r0, ZB, CHUNK2)

        @pl.when(s == NS - 1)
        def _():
            _zero_agg(rows[0], agg, r0, ZLAST, CHUNK2)

        plsc.subcore_barrier()

        _run_chunks(h2, sidx, didx, rows, sems, agg, NCHUNK2)

        plsc.subcore_barrier()

        def copy_out(out_hbm):
            @pl.when(s < NS - 1)
            def _():
                pltpu.sync_copy(agg.at[pl.ds(r0, ZB)], out_hbm.at[pl.ds(r0, ZB)])

            @pl.when(s == NS - 1)
            def _():
                pltpu.sync_copy(agg.at[pl.ds(r0, ZLAST)],
                                out_hbm.at[pl.ds(r0, ZLAST)])

        @pl.when(c == 0)
        def _():
            copy_out(out_a)

        @pl.when(c == 1)
        def _():
            copy_out(out_b)

    return k


def kernel(x, edge_index, W1, b1, W2, b2):
    ei = edge_index.astype(jnp.int32)
    src1 = ei[0].reshape(NS, NCHUNK1, CHUNK1)
    dst1 = ei[1].reshape(NS, NCHUNK1, CHUNK1)
    src2 = ei[0].reshape(NC, NS, NCHUNK2, CHUNK2)
    dst2 = ei[1].reshape(NC, NS, NCHUNK2, CHUNK2)

    lo, hi = _linear1(x, W1, b1.reshape(1, HIDDEN))
    a_lo, a_hi = _scatter1()(lo, hi, src1, dst1)
    h2 = _linear2(a_lo, a_hi, W2[: HIDDEN // 2], W2[HIDDEN // 2:],
                  b2.reshape(1, NUM_CLASSES))
    pa, pb = _scatter2()(h2, src2, dst2)
    return _final_add(pa, pb)


# speedup vs baseline: 1.0351x; 1.0351x over previous
import functools

import jax
import jax.numpy as jnp
from jax import lax
from jax.experimental import pallas as pl
from jax.experimental.pallas import tpu as pltpu
from jax.experimental.pallas import tpu_sc as plsc

N_NODES = 10000
N_EDGES = 160000
IN_FEATS = 256
HIDDEN = 256
NUM_CLASSES = 64

NC = 2
NS = 16
NBUF = 4

CHUNK1 = 40
EPT1 = N_EDGES // NS
NCHUNK1 = EPT1 // CHUNK1

CHUNK2 = 40
EPT2 = N_EDGES // (NC * NS)
NCHUNK2 = EPT2 // CHUNK2

ZB = 632
ZLAST = N_NODES - (NS - 1) * ZB


def _linear1(x, w, b):
    blk = 1000
    half = HIDDEN // 2

    def body(x_ref, w_ref, b_ref, lo_ref, hi_ref):
        h = jnp.dot(x_ref[...], w_ref[...], preferred_element_type=jnp.float32)
        h = h + b_ref[...]
        lo_ref[...] = h[:, :half]
        hi_ref[...] = h[:, half:]

    return pl.pallas_call(
        body,
        grid=(N_NODES // blk,),
        in_specs=[
            pl.BlockSpec((blk, IN_FEATS), lambda i: (i, 0)),
            pl.BlockSpec((IN_FEATS, HIDDEN), lambda i: (0, 0)),
            pl.BlockSpec((1, HIDDEN), lambda i: (0, 0)),
        ],
        out_specs=[
            pl.BlockSpec((blk, half), lambda i: (i, 0)),
            pl.BlockSpec((blk, half), lambda i: (i, 0)),
        ],
        out_shape=[jax.ShapeDtypeStruct((N_NODES, half), jnp.float32)] * 2,
    )(x, w, b)


def _linear2(lo, hi, wa, wb, b):
    blk = 1000

    def body(lo_ref, hi_ref, wa_ref, wb_ref, b_ref, o_ref):
        h = jnp.dot(jnp.maximum(lo_ref[...], 0.0), wa_ref[...],
                    preferred_element_type=jnp.float32)
        h = h + jnp.dot(jnp.maximum(hi_ref[...], 0.0), wb_ref[...],
                        preferred_element_type=jnp.float32)
        o_ref[...] = h + b_ref[...]

    return pl.pallas_call(
        body,
        grid=(N_NODES // blk,),
        in_specs=[
            pl.BlockSpec((blk, HIDDEN // 2), lambda i: (i, 0)),
            pl.BlockSpec((blk, HIDDEN // 2), lambda i: (i, 0)),
            pl.BlockSpec((HIDDEN // 2, NUM_CLASSES), lambda i: (0, 0)),
            pl.BlockSpec((HIDDEN // 2, NUM_CLASSES), lambda i: (0, 0)),
            pl.BlockSpec((1, NUM_CLASSES), lambda i: (0, 0)),
        ],
        out_specs=pl.BlockSpec((blk, NUM_CLASSES), lambda i: (i, 0)),
        out_shape=jax.ShapeDtypeStruct((N_NODES, NUM_CLASSES), jnp.float32),
    )(lo, hi, wa, wb, b)


def _final_add(a, b):
    blk = 1000

    def body(a_ref, b_ref, o_ref):
        o_ref[...] = a_ref[...] + b_ref[...]

    return pl.pallas_call(
        body,
        grid=(N_NODES // blk,),
        in_specs=[
            pl.BlockSpec((blk, NUM_CLASSES), lambda i: (i, 0)),
            pl.BlockSpec((blk, NUM_CLASSES), lambda i: (i, 0)),
        ],
        out_specs=pl.BlockSpec((blk, NUM_CLASSES), lambda i: (i, 0)),
        out_shape=jax.ShapeDtypeStruct((N_NODES, NUM_CLASSES), jnp.float32),
    )(a, b)


def _zero_fill(buf, nrows, d):
    per_row = d // 16

    def body(i, carry):
        r = i // per_row
        cc = lax.rem(i, per_row)
        buf[r, pl.ds(cc * 16, 16)] = jnp.zeros((16,), jnp.float32)
        return carry

    lax.fori_loop(0, nrows * per_row, body, 0)


def _zero_agg(zbuf, agg, r0, nrows, chunk):
    nfull, tail = nrows // chunk, nrows % chunk
    for kk in range(nfull):
        pltpu.sync_copy(zbuf, agg.at[pl.ds(r0 + kk * chunk, chunk)])
    if tail:
        pltpu.sync_copy(zbuf.at[pl.ds(0, tail)],
                        agg.at[pl.ds(r0 + nfull * chunk, tail)])


def _run_chunks(h_hbm, sidx, didx, rows, gsems, ssem, agg, nchunk):
    def chunk_body(jj, carry):
        base = NBUF * jj
        gs = [pltpu.async_copy(h_hbm.at[sidx.at[base + b]], rows[b], gsems[b])
              for b in range(NBUF)]
        ss = []
        for b in range(NBUF):
            gs[b].wait()
            ss.append(pltpu.async_copy(rows[b], agg.at[didx.at[base + b]],
                                       ssem, add=True))
        for sd in ss:
            sd.wait()
        return carry

    lax.fori_loop(0, nchunk // NBUF, chunk_body, 0)
    for j in range(nchunk - nchunk % NBUF, nchunk):
        g = pltpu.async_copy(h_hbm.at[sidx.at[j]], rows[0], gsems[0])
        g.wait()
        pltpu.sync_copy(rows[0], agg.at[didx.at[j]], add=True)


def _scatter1():
    d = HIDDEN // 2
    mesh = plsc.VectorSubcoreMesh(core_axis_name="c", subcore_axis_name="s")

    @functools.partial(
        pl.kernel,
        out_type=[jax.ShapeDtypeStruct((N_NODES, d), jnp.float32)] * 2,
        mesh=mesh,
        scratch_types=[
            pltpu.VMEM((NCHUNK1, CHUNK1), jnp.int32),
            pltpu.VMEM((NCHUNK1, CHUNK1), jnp.int32),
            [pltpu.VMEM((CHUNK1, d), jnp.float32)] * NBUF,
            pltpu.VMEM_SHARED((N_NODES, d), jnp.float32),
            [pltpu.SemaphoreType.DMA] * NBUF,
            pltpu.SemaphoreType.DMA,
        ],
        compiler_params=pltpu.CompilerParams(use_tc_tiling_on_sc=False),
    )
    def k(h_lo, h_hi, src_hbm, dst_hbm, out_lo, out_hi,
          sidx, didx, rows, agg, gsems, ssem):
        c = lax.axis_index("c")
        s = lax.axis_index("s")
        r0 = pl.multiple_of(s * ZB, 8)

        pltpu.sync_copy(src_hbm.at[s], sidx)
        pltpu.sync_copy(dst_hbm.at[s], didx)

        _zero_fill(rows[0], CHUNK1, d)

        @pl.when(s < NS - 1)
        def _():
            _zero_agg(rows[0], agg, r0, ZB, CHUNK1)

        @pl.when(s == NS - 1)
        def _():
            _zero_agg(rows[0], agg, r0, ZLAST, CHUNK1)

        plsc.subcore_barrier()

        @pl.when(c == 0)
        def _():
            _run_chunks(h_lo, sidx, didx, rows, gsems, ssem, agg, NCHUNK1)

        @pl.when(c == 1)
        def _():
            _run_chunks(h_hi, sidx, didx, rows, gsems, ssem, agg, NCHUNK1)

        plsc.subcore_barrier()

        def copy_out(out_hbm):
            @pl.when(s < NS - 1)
            def _():
                pltpu.sync_copy(agg.at[pl.ds(r0, ZB)], out_hbm.at[pl.ds(r0, ZB)])

            @pl.when(s == NS - 1)
            def _():
                pltpu.sync_copy(agg.at[pl.ds(r0, ZLAST)],
                                out_hbm.at[pl.ds(r0, ZLAST)])

        @pl.when(c == 0)
        def _():
            copy_out(out_lo)

        @pl.when(c == 1)
        def _():
            copy_out(out_hi)

    return k


def _scatter2():
    d = NUM_CLASSES
    mesh = plsc.VectorSubcoreMesh(core_axis_name="c", subcore_axis_name="s")

    @functools.partial(
        pl.kernel,
        out_type=[jax.ShapeDtypeStruct((N_NODES, d), jnp.float32)] * 2,
        mesh=mesh,
        scratch_types=[
            pltpu.VMEM((NCHUNK2, CHUNK2), jnp.int32),
            pltpu.VMEM((NCHUNK2, CHUNK2), jnp.int32),
            [pltpu.VMEM((CHUNK2, d), jnp.float32)] * NBUF,
            pltpu.VMEM_SHARED((N_NODES, d), jnp.float32),
            [pltpu.SemaphoreType.DMA] * NBUF,
            pltpu.SemaphoreType.DMA,
        ],
        compiler_params=pltpu.CompilerParams(use_tc_tiling_on_sc=False),
    )
    def k(h2, src_hbm, dst_hbm, out_a, out_b,
          sidx, didx, rows, agg, gsems, ssem):
        c = lax.axis_index("c")
        s = lax.axis_index("s")
        r0 = pl.multiple_of(s * ZB, 8)

        pltpu.sync_copy(src_hbm.at[c, s], sidx)
        pltpu.sync_copy(dst_hbm.at[c, s], didx)

        _zero_fill(rows[0], CHUNK2, d)

        @pl.when(s < NS - 1)
        def _():
            _zero_agg(rows[0], agg, r0, ZB, CHUNK2)

        @pl.when(s == NS - 1)
        def _():
            _zero_agg(rows[0], agg, r0, ZLAST, CHUNK2)

        plsc.subcore_barrier()

        _run_chunks(h2, sidx, didx, rows, gsems, ssem, agg, NCHUNK2)

        plsc.subcore_barrier()

        def copy_out(out_hbm):
            @pl.when(s < NS - 1)
            def _():
                pltpu.sync_copy(agg.at[pl.ds(r0, ZB)], out_hbm.at[pl.ds(r0, ZB)])

            @pl.when(s == NS - 1)
            def _():
                pltpu.sync_copy(agg.at[pl.ds(r0, ZLAST)],
                                out_hbm.at[pl.ds(r0, ZLAST)])

        @pl.when(c == 0)
        def _():
            copy_out(out_a)

        @pl.when(c == 1)
        def _():
            copy_out(out_b)

    return k


def kernel(x, edge_index, W1, b1, W2, b2):
    ei = edge_index.astype(jnp.int32)
    src1 = ei[0].reshape(NS, NCHUNK1, CHUNK1)
    dst1 = ei[1].reshape(NS, NCHUNK1, CHUNK1)
    src2 = ei[0].reshape(NC, NS, NCHUNK2, CHUNK2)
    dst2 = ei[1].reshape(NC, NS, NCHUNK2, CHUNK2)

    lo, hi = _linear1(x, W1, b1.reshape(1, HIDDEN))
    a_lo, a_hi = _scatter1()(lo, hi, src1, dst1)
    h2 = _linear2(a_lo, a_hi, W2[: HIDDEN // 2], W2[HIDDEN // 2:],
                  b2.reshape(1, NUM_CLASSES))
    pa, pb = _scatter2()(h2, src2, dst2)
    return _final_add(pa, pb)


# speedup vs baseline: 1.0837x; 1.0469x over previous
import functools

import jax
import jax.numpy as jnp
from jax import lax
from jax.experimental import pallas as pl
from jax.experimental.pallas import tpu as pltpu
from jax.experimental.pallas import tpu_sc as plsc

N_NODES = 10000
N_EDGES = 160000
IN_FEATS = 256
HIDDEN = 256
NUM_CLASSES = 64

NC = 2
NS = 16
NBUF1 = 3
NBUF2 = 8

CHUNK1 = 80
EPT1 = N_EDGES // NS
NCHUNK1 = EPT1 // CHUNK1

CHUNK2 = 40
EPT2 = N_EDGES // (NC * NS)
NCHUNK2 = EPT2 // CHUNK2

ZB = 632
ZLAST = N_NODES - (NS - 1) * ZB


def _linear1(x, w, b):
    blk = 1000
    half = HIDDEN // 2

    def body(x_ref, w_ref, b_ref, lo_ref, hi_ref):
        h = jnp.dot(x_ref[...], w_ref[...], preferred_element_type=jnp.float32)
        h = h + b_ref[...]
        lo_ref[...] = h[:, :half]
        hi_ref[...] = h[:, half:]

    return pl.pallas_call(
        body,
        grid=(N_NODES // blk,),
        in_specs=[
            pl.BlockSpec((blk, IN_FEATS), lambda i: (i, 0)),
            pl.BlockSpec((IN_FEATS, HIDDEN), lambda i: (0, 0)),
            pl.BlockSpec((1, HIDDEN), lambda i: (0, 0)),
        ],
        out_specs=[
            pl.BlockSpec((blk, half), lambda i: (i, 0)),
            pl.BlockSpec((blk, half), lambda i: (i, 0)),
        ],
        out_shape=[jax.ShapeDtypeStruct((N_NODES, half), jnp.float32)] * 2,
    )(x, w, b)


def _linear2(lo, hi, wa, wb, b):
    blk = 1000

    def body(lo_ref, hi_ref, wa_ref, wb_ref, b_ref, o_ref):
        h = jnp.dot(jnp.maximum(lo_ref[...], 0.0), wa_ref[...],
                    preferred_element_type=jnp.float32)
        h = h + jnp.dot(jnp.maximum(hi_ref[...], 0.0), wb_ref[...],
                        preferred_element_type=jnp.float32)
        o_ref[...] = h + b_ref[...]

    return pl.pallas_call(
        body,
        grid=(N_NODES // blk,),
        in_specs=[
            pl.BlockSpec((blk, HIDDEN // 2), lambda i: (i, 0)),
            pl.BlockSpec((blk, HIDDEN // 2), lambda i: (i, 0)),
            pl.BlockSpec((HIDDEN // 2, NUM_CLASSES), lambda i: (0, 0)),
            pl.BlockSpec((HIDDEN // 2, NUM_CLASSES), lambda i: (0, 0)),
            pl.BlockSpec((1, NUM_CLASSES), lambda i: (0, 0)),
        ],
        out_specs=pl.BlockSpec((blk, NUM_CLASSES), lambda i: (i, 0)),
        out_shape=jax.ShapeDtypeStruct((N_NODES, NUM_CLASSES), jnp.float32),
    )(lo, hi, wa, wb, b)


def _final_add(a, b):
    blk = 1000

    def body(a_ref, b_ref, o_ref):
        o_ref[...] = a_ref[...] + b_ref[...]

    return pl.pallas_call(
        body,
        grid=(N_NODES // blk,),
        in_specs=[
            pl.BlockSpec((blk, NUM_CLASSES), lambda i: (i, 0)),
            pl.BlockSpec((blk, NUM_CLASSES), lambda i: (i, 0)),
        ],
        out_specs=pl.BlockSpec((blk, NUM_CLASSES), lambda i: (i, 0)),
        out_shape=jax.ShapeDtypeStruct((N_NODES, NUM_CLASSES), jnp.float32),
    )(a, b)


def _zero_fill(buf, nrows, d):
    per_row = d // 16

    def body(i, carry):
        r = i // per_row
        cc = lax.rem(i, per_row)
        buf[r, pl.ds(cc * 16, 16)] = jnp.zeros((16,), jnp.float32)
        return carry

    lax.fori_loop(0, nrows * per_row, body, 0)


def _zero_agg(zbuf, agg, r0, nrows, chunk):
    nfull, tail = nrows // chunk, nrows % chunk
    for kk in range(nfull):
        pltpu.sync_copy(zbuf, agg.at[pl.ds(r0 + kk * chunk, chunk)])
    if tail:
        pltpu.sync_copy(zbuf.at[pl.ds(0, tail)],
                        agg.at[pl.ds(r0 + nfull * chunk, tail)])


def _run_chunks(h_hbm, sidx, didx, rows, gsems, ssem, agg, nchunk):
    nb = len(rows)

    def chunk_body(jj, carry):
        base = nb * jj
        gs = [pltpu.async_copy(h_hbm.at[sidx.at[base + b]], rows[b], gsems[b])
              for b in range(nb)]
        ss = []
        for b in range(nb):
            gs[b].wait()
            ss.append(pltpu.async_copy(rows[b], agg.at[didx.at[base + b]],
                                       ssem, add=True))
        for sd in ss:
            sd.wait()
        return carry

    lax.fori_loop(0, nchunk // nb, chunk_body, 0)
    for j in range(nchunk - nchunk % nb, nchunk):
        g = pltpu.async_copy(h_hbm.at[sidx.at[j]], rows[0], gsems[0])
        g.wait()
        pltpu.sync_copy(rows[0], agg.at[didx.at[j]], add=True)


def _scatter1():
    d = HIDDEN // 2
    mesh = plsc.VectorSubcoreMesh(core_axis_name="c", subcore_axis_name="s")

    @functools.partial(
        pl.kernel,
        out_type=[jax.ShapeDtypeStruct((N_NODES, d), jnp.float32)] * 2,
        mesh=mesh,
        scratch_types=[
            pltpu.VMEM((NCHUNK1, CHUNK1), jnp.int32),
            pltpu.VMEM((NCHUNK1, CHUNK1), jnp.int32),
            [pltpu.VMEM((CHUNK1, d), jnp.float32)] * NBUF1,
            pltpu.VMEM_SHARED((N_NODES, d), jnp.float32),
            [pltpu.SemaphoreType.DMA] * NBUF1,
            pltpu.SemaphoreType.DMA,
        ],
        compiler_params=pltpu.CompilerParams(use_tc_tiling_on_sc=False),
    )
    def k(h_lo, h_hi, src_hbm, dst_hbm, out_lo, out_hi,
          sidx, didx, rows, agg, gsems, ssem):
        c = lax.axis_index("c")
        s = lax.axis_index("s")
        r0 = pl.multiple_of(s * ZB, 8)

        pltpu.sync_copy(src_hbm.at[s], sidx)
        pltpu.sync_copy(dst_hbm.at[s], didx)

        _zero_fill(rows[0], CHUNK1, d)

        @pl.when(s < NS - 1)
        def _():
            _zero_agg(rows[0], agg, r0, ZB, CHUNK1)

        @pl.when(s == NS - 1)
        def _():
            _zero_agg(rows[0], agg, r0, ZLAST, CHUNK1)

        plsc.subcore_barrier()

        @pl.when(c == 0)
        def _():
            _run_chunks(h_lo, sidx, didx, rows, gsems, ssem, agg, NCHUNK1)

        @pl.when(c == 1)
        def _():
            _run_chunks(h_hi, sidx, didx, rows, gsems, ssem, agg, NCHUNK1)

        plsc.subcore_barrier()

        def copy_out(out_hbm):
            @pl.when(s < NS - 1)
            def _():
                pltpu.sync_copy(agg.at[pl.ds(r0, ZB)], out_hbm.at[pl.ds(r0, ZB)])

            @pl.when(s == NS - 1)
            def _():
                pltpu.sync_copy(agg.at[pl.ds(r0, ZLAST)],
                                out_hbm.at[pl.ds(r0, ZLAST)])

        @pl.when(c == 0)
        def _():
            copy_out(out_lo)

        @pl.when(c == 1)
        def _():
            copy_out(out_hi)

    return k


def _scatter2():
    d = NUM_CLASSES
    mesh = plsc.VectorSubcoreMesh(core_axis_name="c", subcore_axis_name="s")

    @functools.partial(
        pl.kernel,
        out_type=[jax.ShapeDtypeStruct((N_NODES, d), jnp.float32)] * 2,
        mesh=mesh,
        scratch_types=[
            pltpu.VMEM((NCHUNK2, CHUNK2), jnp.int32),
            pltpu.VMEM((NCHUNK2, CHUNK2), jnp.int32),
            [pltpu.VMEM((CHUNK2, d), jnp.float32)] * NBUF2,
            pltpu.VMEM_SHARED((N_NODES, d), jnp.float32),
            [pltpu.SemaphoreType.DMA] * NBUF2,
            pltpu.SemaphoreType.DMA,
        ],
        compiler_params=pltpu.CompilerParams(use_tc_tiling_on_sc=False),
    )
    def k(h2, src_hbm, dst_hbm, out_a, out_b,
          sidx, didx, rows, agg, gsems, ssem):
        c = lax.axis_index("c")
        s = lax.axis_index("s")
        r0 = pl.multiple_of(s * ZB, 8)

        pltpu.sync_copy(src_hbm.at[c, s], sidx)
        pltpu.sync_copy(dst_hbm.at[c, s], didx)

        _zero_fill(rows[0], CHUNK2, d)

        @pl.when(s < NS - 1)
        def _():
            _zero_agg(rows[0], agg, r0, ZB, CHUNK2)

        @pl.when(s == NS - 1)
        def _():
            _zero_agg(rows[0], agg, r0, ZLAST, CHUNK2)

        plsc.subcore_barrier()

        _run_chunks(h2, sidx, didx, rows, gsems, ssem, agg, NCHUNK2)

        plsc.subcore_barrier()

        def copy_out(out_hbm):
            @pl.when(s < NS - 1)
            def _():
                pltpu.sync_copy(agg.at[pl.ds(r0, ZB)], out_hbm.at[pl.ds(r0, ZB)])

            @pl.when(s == NS - 1)
            def _():
                pltpu.sync_copy(agg.at[pl.ds(r0, ZLAST)],
                                out_hbm.at[pl.ds(r0, ZLAST)])

        @pl.when(c == 0)
        def _():
            copy_out(out_a)

        @pl.when(c == 1)
        def _():
            copy_out(out_b)

    return k


def kernel(x, edge_index, W1, b1, W2, b2):
    ei = edge_index.astype(jnp.int32)
    src1 = ei[0].reshape(NS, NCHUNK1, CHUNK1)
    dst1 = ei[1].reshape(NS, NCHUNK1, CHUNK1)
    src2 = ei[0].reshape(NC, NS, NCHUNK2, CHUNK2)
    dst2 = ei[1].reshape(NC, NS, NCHUNK2, CHUNK2)

    lo, hi = _linear1(x, W1, b1.reshape(1, HIDDEN))
    a_lo, a_hi = _scatter1()(lo, hi, src1, dst1)
    h2 = _linear2(a_lo, a_hi, W2[: HIDDEN // 2], W2[HIDDEN // 2:],
                  b2.reshape(1, NUM_CLASSES))
    pa, pb = _scatter2()(h2, src2, dst2)
    return _final_add(pa, pb)


# speedup vs baseline: 1.1266x; 1.0396x over previous
import functools

import jax
import jax.numpy as jnp
from jax import lax
from jax.experimental import pallas as pl
from jax.experimental.pallas import tpu as pltpu
from jax.experimental.pallas import tpu_sc as plsc

N_NODES = 10000
N_EDGES = 160000
IN_FEATS = 256
HIDDEN = 256
NUM_CLASSES = 64

NC = 2
NS = 16
NBUF1 = 6
NBUF2 = 12

CHUNK1 = 40
EPT1 = N_EDGES // NS
NCHUNK1 = EPT1 // CHUNK1

CHUNK2 = 40
EPT2 = N_EDGES // (NC * NS)
NCHUNK2 = EPT2 // CHUNK2

ZB = 632
ZLAST = N_NODES - (NS - 1) * ZB


def _linear1(x, w, b):
    blk = 1000
    half = HIDDEN // 2

    def body(x_ref, w_ref, b_ref, lo_ref, hi_ref):
        h = jnp.dot(x_ref[...], w_ref[...], preferred_element_type=jnp.float32)
        h = h + b_ref[...]
        lo_ref[...] = h[:, :half]
        hi_ref[...] = h[:, half:]

    return pl.pallas_call(
        body,
        grid=(N_NODES // blk,),
        in_specs=[
            pl.BlockSpec((blk, IN_FEATS), lambda i: (i, 0)),
            pl.BlockSpec((IN_FEATS, HIDDEN), lambda i: (0, 0)),
            pl.BlockSpec((1, HIDDEN), lambda i: (0, 0)),
        ],
        out_specs=[
            pl.BlockSpec((blk, half), lambda i: (i, 0)),
            pl.BlockSpec((blk, half), lambda i: (i, 0)),
        ],
        out_shape=[jax.ShapeDtypeStruct((N_NODES, half), jnp.float32)] * 2,
    )(x, w, b)


def _linear2(lo, hi, wa, wb, b):
    blk = 1000

    def body(lo_ref, hi_ref, wa_ref, wb_ref, b_ref, o_ref):
        h = jnp.dot(jnp.maximum(lo_ref[...], 0.0), wa_ref[...],
                    preferred_element_type=jnp.float32)
        h = h + jnp.dot(jnp.maximum(hi_ref[...], 0.0), wb_ref[...],
                        preferred_element_type=jnp.float32)
        o_ref[...] = h + b_ref[...]

    return pl.pallas_call(
        body,
        grid=(N_NODES // blk,),
        in_specs=[
            pl.BlockSpec((blk, HIDDEN // 2), lambda i: (i, 0)),
            pl.BlockSpec((blk, HIDDEN // 2), lambda i: (i, 0)),
            pl.BlockSpec((HIDDEN // 2, NUM_CLASSES), lambda i: (0, 0)),
            pl.BlockSpec((HIDDEN // 2, NUM_CLASSES), lambda i: (0, 0)),
            pl.BlockSpec((1, NUM_CLASSES), lambda i: (0, 0)),
        ],
        out_specs=pl.BlockSpec((blk, NUM_CLASSES), lambda i: (i, 0)),
        out_shape=jax.ShapeDtypeStruct((N_NODES, NUM_CLASSES), jnp.float32),
    )(lo, hi, wa, wb, b)


def _final_add(a, b):
    blk = 1000

    def body(a_ref, b_ref, o_ref):
        o_ref[...] = a_ref[...] + b_ref[...]

    return pl.pallas_call(
        body,
        grid=(N_NODES // blk,),
        in_specs=[
            pl.BlockSpec((blk, NUM_CLASSES), lambda i: (i, 0)),
            pl.BlockSpec((blk, NUM_CLASSES), lambda i: (i, 0)),
        ],
        out_specs=pl.BlockSpec((blk, NUM_CLASSES), lambda i: (i, 0)),
        out_shape=jax.ShapeDtypeStruct((N_NODES, NUM_CLASSES), jnp.float32),
    )(a, b)


def _zero_fill(buf, nrows, d):
    per_row = d // 16

    def body(i, carry):
        r = i // per_row
        cc = lax.rem(i, per_row)
        buf[r, pl.ds(cc * 16, 16)] = jnp.zeros((16,), jnp.float32)
        return carry

    lax.fori_loop(0, nrows * per_row, body, 0)


def _zero_agg(zbuf, agg, r0, nrows, chunk):
    nfull, tail = nrows // chunk, nrows % chunk
    for kk in range(nfull):
        pltpu.sync_copy(zbuf, agg.at[pl.ds(r0 + kk * chunk, chunk)])
    if tail:
        pltpu.sync_copy(zbuf.at[pl.ds(0, tail)],
                        agg.at[pl.ds(r0 + nfull * chunk, tail)])


def _run_chunks(h_hbm, sidx, didx, rows, gsems, ssem, agg, nchunk):
    nb = len(rows)

    def chunk_body(jj, carry):
        base = nb * jj
        gs = [pltpu.async_copy(h_hbm.at[sidx.at[base + b]], rows[b], gsems[b])
              for b in range(nb)]
        ss = []
        for b in range(nb):
            gs[b].wait()
            ss.append(pltpu.async_copy(rows[b], agg.at[didx.at[base + b]],
                                       ssem, add=True))
        for sd in ss:
            sd.wait()
        return carry

    lax.fori_loop(0, nchunk // nb, chunk_body, 0)
    for j in range(nchunk - nchunk % nb, nchunk):
        g = pltpu.async_copy(h_hbm.at[sidx.at[j]], rows[0], gsems[0])
        g.wait()
        pltpu.sync_copy(rows[0], agg.at[didx.at[j]], add=True)


def _scatter1():
    d = HIDDEN // 2
    mesh = plsc.VectorSubcoreMesh(core_axis_name="c", subcore_axis_name="s")

    @functools.partial(
        pl.kernel,
        out_type=[jax.ShapeDtypeStruct((N_NODES, d), jnp.float32)] * 2,
        mesh=mesh,
        scratch_types=[
            pltpu.VMEM((NCHUNK1, CHUNK1), jnp.int32),
            pltpu.VMEM((NCHUNK1, CHUNK1), jnp.int32),
            [pltpu.VMEM((CHUNK1, d), jnp.float32)] * NBUF1,
            pltpu.VMEM_SHARED((N_NODES, d), jnp.float32),
            [pltpu.SemaphoreType.DMA] * NBUF1,
            pltpu.SemaphoreType.DMA,
        ],
        compiler_params=pltpu.CompilerParams(use_tc_tiling_on_sc=False),
    )
    def k(h_lo, h_hi, src_hbm, dst_hbm, out_lo, out_hi,
          sidx, didx, rows, agg, gsems, ssem):
        c = lax.axis_index("c")
        s = lax.axis_index("s")
        r0 = pl.multiple_of(s * ZB, 8)

        pltpu.sync_copy(src_hbm.at[s], sidx)
        pltpu.sync_copy(dst_hbm.at[s], didx)

        _zero_fill(rows[0], CHUNK1, d)

        @pl.when(s < NS - 1)
        def _():
            _zero_agg(rows[0], agg, r0, ZB, CHUNK1)

        @pl.when(s == NS - 1)
        def _():
            _zero_agg(rows[0], agg, r0, ZLAST, CHUNK1)

        plsc.subcore_barrier()

        @pl.when(c == 0)
        def _():
            _run_chunks(h_lo, sidx, didx, rows, gsems, ssem, agg, NCHUNK1)

        @pl.when(c == 1)
        def _():
            _run_chunks(h_hi, sidx, didx, rows, gsems, ssem, agg, NCHUNK1)

        plsc.subcore_barrier()

        def copy_out(out_hbm):
            @pl.when(s < NS - 1)
            def _():
                pltpu.sync_copy(agg.at[pl.ds(r0, ZB)], out_hbm.at[pl.ds(r0, ZB)])

            @pl.when(s == NS - 1)
            def _():
                pltpu.sync_copy(agg.at[pl.ds(r0, ZLAST)],
                                out_hbm.at[pl.ds(r0, ZLAST)])

        @pl.when(c == 0)
        def _():
            copy_out(out_lo)

        @pl.when(c == 1)
        def _():
            copy_out(out_hi)

    return k


def _scatter2():
    d = NUM_CLASSES
    mesh = plsc.VectorSubcoreMesh(core_axis_name="c", subcore_axis_name="s")

    @functools.partial(
        pl.kernel,
        out_type=[jax.ShapeDtypeStruct((N_NODES, d), jnp.float32)] * 2,
        mesh=mesh,
        scratch_types=[
            pltpu.VMEM((NCHUNK2, CHUNK2), jnp.int32),
            pltpu.VMEM((NCHUNK2, CHUNK2), jnp.int32),
            [pltpu.VMEM((CHUNK2, d), jnp.float32)] * NBUF2,
            pltpu.VMEM_SHARED((N_NODES, d), jnp.float32),
            [pltpu.SemaphoreType.DMA] * NBUF2,
            pltpu.SemaphoreType.DMA,
        ],
        compiler_params=pltpu.CompilerParams(use_tc_tiling_on_sc=False),
    )
    def k(h2, src_hbm, dst_hbm, out_a, out_b,
          sidx, didx, rows, agg, gsems, ssem):
        c = lax.axis_index("c")
        s = lax.axis_index("s")
        r0 = pl.multiple_of(s * ZB, 8)

        pltpu.sync_copy(src_hbm.at[c, s], sidx)
        pltpu.sync_copy(dst_hbm.at[c, s], didx)

        _zero_fill(rows[0], CHUNK2, d)

        @pl.when(s < NS - 1)
        def _():
            _zero_agg(rows[0], agg, r0, ZB, CHUNK2)

        @pl.when(s == NS - 1)
        def _():
            _zero_agg(rows[0], agg, r0, ZLAST, CHUNK2)

        plsc.subcore_barrier()

        _run_chunks(h2, sidx, didx, rows, gsems, ssem, agg, NCHUNK2)

        plsc.subcore_barrier()

        def copy_out(out_hbm):
            @pl.when(s < NS - 1)
            def _():
                pltpu.sync_copy(agg.at[pl.ds(r0, ZB)], out_hbm.at[pl.ds(r0, ZB)])

            @pl.when(s == NS - 1)
            def _():
                pltpu.sync_copy(agg.at[pl.ds(r0, ZLAST)],
                                out_hbm.at[pl.ds(r0, ZLAST)])

        @pl.when(c == 0)
        def _():
            copy_out(out_a)

        @pl.when(c == 1)
        def _():
            copy_out(out_b)

    return k


def kernel(x, edge_index, W1, b1, W2, b2):
    ei = edge_index.astype(jnp.int32)
    src1 = ei[0].reshape(NS, NCHUNK1, CHUNK1)
    dst1 = ei[1].reshape(NS, NCHUNK1, CHUNK1)
    src2 = ei[0].reshape(NC, NS, NCHUNK2, CHUNK2)
    dst2 = ei[1].reshape(NC, NS, NCHUNK2, CHUNK2)

    lo, hi = _linear1(x, W1, b1.reshape(1, HIDDEN))
    a_lo, a_hi = _scatter1()(lo, hi, src1, dst1)
    h2 = _linear2(a_lo, a_hi, W2[: HIDDEN // 2], W2[HIDDEN // 2:],
                  b2.reshape(1, NUM_CLASSES))
    pa, pb = _scatter2()(h2, src2, dst2)
    return _final_add(pa, pb)


# speedup vs baseline: 1.1464x; 1.0175x over previous
import functools

import jax
import jax.numpy as jnp
from jax import lax
from jax.experimental import pallas as pl
from jax.experimental.pallas import tpu as pltpu
from jax.experimental.pallas import tpu_sc as plsc

N_NODES = 10000
N_EDGES = 160000
IN_FEATS = 256
HIDDEN = 256
NUM_CLASSES = 64

NC = 2
NS = 16
NBUF1 = 12
NBUF2 = 16

CHUNK1 = 16
EPT1 = N_EDGES // NS
NCHUNK1 = EPT1 // CHUNK1

CHUNK2 = 40
EPT2 = N_EDGES // (NC * NS)
NCHUNK2 = EPT2 // CHUNK2

ZB = 632
ZLAST = N_NODES - (NS - 1) * ZB


def _linear1(x, w, b):
    blk = 1000
    half = HIDDEN // 2

    def body(x_ref, w_ref, b_ref, lo_ref, hi_ref):
        h = jnp.dot(x_ref[...], w_ref[...], preferred_element_type=jnp.float32)
        h = h + b_ref[...]
        lo_ref[...] = h[:, :half]
        hi_ref[...] = h[:, half:]

    return pl.pallas_call(
        body,
        grid=(N_NODES // blk,),
        in_specs=[
            pl.BlockSpec((blk, IN_FEATS), lambda i: (i, 0)),
            pl.BlockSpec((IN_FEATS, HIDDEN), lambda i: (0, 0)),
            pl.BlockSpec((1, HIDDEN), lambda i: (0, 0)),
        ],
        out_specs=[
            pl.BlockSpec((blk, half), lambda i: (i, 0)),
            pl.BlockSpec((blk, half), lambda i: (i, 0)),
        ],
        out_shape=[jax.ShapeDtypeStruct((N_NODES, half), jnp.float32)] * 2,
    )(x, w, b)


def _linear2(lo, hi, wa, wb, b):
    blk = 1000

    def body(lo_ref, hi_ref, wa_ref, wb_ref, b_ref, o_ref):
        h = jnp.dot(jnp.maximum(lo_ref[...], 0.0), wa_ref[...],
                    preferred_element_type=jnp.float32)
        h = h + jnp.dot(jnp.maximum(hi_ref[...], 0.0), wb_ref[...],
                        preferred_element_type=jnp.float32)
        o_ref[...] = h + b_ref[...]

    return pl.pallas_call(
        body,
        grid=(N_NODES // blk,),
        in_specs=[
            pl.BlockSpec((blk, HIDDEN // 2), lambda i: (i, 0)),
            pl.BlockSpec((blk, HIDDEN // 2), lambda i: (i, 0)),
            pl.BlockSpec((HIDDEN // 2, NUM_CLASSES), lambda i: (0, 0)),
            pl.BlockSpec((HIDDEN // 2, NUM_CLASSES), lambda i: (0, 0)),
            pl.BlockSpec((1, NUM_CLASSES), lambda i: (0, 0)),
        ],
        out_specs=pl.BlockSpec((blk, NUM_CLASSES), lambda i: (i, 0)),
        out_shape=jax.ShapeDtypeStruct((N_NODES, NUM_CLASSES), jnp.float32),
    )(lo, hi, wa, wb, b)


def _final_add(a, b):
    blk = 1000

    def body(a_ref, b_ref, o_ref):
        o_ref[...] = a_ref[...] + b_ref[...]

    return pl.pallas_call(
        body,
        grid=(N_NODES // blk,),
        in_specs=[
            pl.BlockSpec((blk, NUM_CLASSES), lambda i: (i, 0)),
            pl.BlockSpec((blk, NUM_CLASSES), lambda i: (i, 0)),
        ],
        out_specs=pl.BlockSpec((blk, NUM_CLASSES), lambda i: (i, 0)),
        out_shape=jax.ShapeDtypeStruct((N_NODES, NUM_CLASSES), jnp.float32),
    )(a, b)


def _zero_fill(buf, nrows, d):
    per_row = d // 16

    def body(i, carry):
        r = i // per_row
        cc = lax.rem(i, per_row)
        buf[r, pl.ds(cc * 16, 16)] = jnp.zeros((16,), jnp.float32)
        return carry

    lax.fori_loop(0, nrows * per_row, body, 0)


def _zero_agg(zbuf, agg, r0, nrows, chunk):
    nfull, tail = nrows // chunk, nrows % chunk
    for kk in range(nfull):
        pltpu.sync_copy(zbuf, agg.at[pl.ds(r0 + kk * chunk, chunk)])
    if tail:
        pltpu.sync_copy(zbuf.at[pl.ds(0, tail)],
                        agg.at[pl.ds(r0 + nfull * chunk, tail)])


def _run_chunks(h_hbm, sidx, didx, rows, gsems, ssem, agg, nchunk):
    nb = len(rows)

    def chunk_body(jj, carry):
        base = nb * jj
        gs = [pltpu.async_copy(h_hbm.at[sidx.at[base + b]], rows[b], gsems[b])
              for b in range(nb)]
        ss = []
        for b in range(nb):
            gs[b].wait()
            ss.append(pltpu.async_copy(rows[b], agg.at[didx.at[base + b]],
                                       ssem, add=True))
        for sd in ss:
            sd.wait()
        return carry

    lax.fori_loop(0, nchunk // nb, chunk_body, 0)
    for j in range(nchunk - nchunk % nb, nchunk):
        g = pltpu.async_copy(h_hbm.at[sidx.at[j]], rows[0], gsems[0])
        g.wait()
        pltpu.sync_copy(rows[0], agg.at[didx.at[j]], add=True)


def _scatter1():
    d = HIDDEN // 2
    mesh = plsc.VectorSubcoreMesh(core_axis_name="c", subcore_axis_name="s")

    @functools.partial(
        pl.kernel,
        out_type=[jax.ShapeDtypeStruct((N_NODES, d), jnp.float32)] * 2,
        mesh=mesh,
        scratch_types=[
            pltpu.VMEM((NCHUNK1, CHUNK1), jnp.int32),
            pltpu.VMEM((NCHUNK1, CHUNK1), jnp.int32),
            [pltpu.VMEM((CHUNK1, d), jnp.float32)] * NBUF1,
            pltpu.VMEM_SHARED((N_NODES, d), jnp.float32),
            [pltpu.SemaphoreType.DMA] * NBUF1,
            pltpu.SemaphoreType.DMA,
        ],
        compiler_params=pltpu.CompilerParams(use_tc_tiling_on_sc=False),
    )
    def k(h_lo, h_hi, src_hbm, dst_hbm, out_lo, out_hi,
          sidx, didx, rows, agg, gsems, ssem):
        c = lax.axis_index("c")
        s = lax.axis_index("s")
        r0 = pl.multiple_of(s * ZB, 8)

        pltpu.sync_copy(src_hbm.at[s], sidx)
        pltpu.sync_copy(dst_hbm.at[s], didx)

        _zero_fill(rows[0], CHUNK1, d)

        @pl.when(s < NS - 1)
        def _():
            _zero_agg(rows[0], agg, r0, ZB, CHUNK1)

        @pl.when(s == NS - 1)
        def _():
            _zero_agg(rows[0], agg, r0, ZLAST, CHUNK1)

        plsc.subcore_barrier()

        @pl.when(c == 0)
        def _():
            _run_chunks(h_lo, sidx, didx, rows, gsems, ssem, agg, NCHUNK1)

        @pl.when(c == 1)
        def _():
            _run_chunks(h_hi, sidx, didx, rows, gsems, ssem, agg, NCHUNK1)

        plsc.subcore_barrier()

        def copy_out(out_hbm):
            @pl.when(s < NS - 1)
            def _():
                pltpu.sync_copy(agg.at[pl.ds(r0, ZB)], out_hbm.at[pl.ds(r0, ZB)])

            @pl.when(s == NS - 1)
            def _():
                pltpu.sync_copy(agg.at[pl.ds(r0, ZLAST)],
                                out_hbm.at[pl.ds(r0, ZLAST)])

        @pl.when(c == 0)
        def _():
            copy_out(out_lo)

        @pl.when(c == 1)
        def _():
            copy_out(out_hi)

    return k


def _scatter2():
    d = NUM_CLASSES
    mesh = plsc.VectorSubcoreMesh(core_axis_name="c", subcore_axis_name="s")

    @functools.partial(
        pl.kernel,
        out_type=[jax.ShapeDtypeStruct((N_NODES, d), jnp.float32)] * 2,
        mesh=mesh,
        scratch_types=[
            pltpu.VMEM((NCHUNK2, CHUNK2), jnp.int32),
            pltpu.VMEM((NCHUNK2, CHUNK2), jnp.int32),
            [pltpu.VMEM((CHUNK2, d), jnp.float32)] * NBUF2,
            pltpu.VMEM_SHARED((N_NODES, d), jnp.float32),
            [pltpu.SemaphoreType.DMA] * NBUF2,
            pltpu.SemaphoreType.DMA,
        ],
        compiler_params=pltpu.CompilerParams(use_tc_tiling_on_sc=False),
    )
    def k(h2, src_hbm, dst_hbm, out_a, out_b,
          sidx, didx, rows, agg, gsems, ssem):
        c = lax.axis_index("c")
        s = lax.axis_index("s")
        r0 = pl.multiple_of(s * ZB, 8)

        pltpu.sync_copy(src_hbm.at[c, s], sidx)
        pltpu.sync_copy(dst_hbm.at[c, s], didx)

        _zero_fill(rows[0], CHUNK2, d)

        @pl.when(s < NS - 1)
        def _():
            _zero_agg(rows[0], agg, r0, ZB, CHUNK2)

        @pl.when(s == NS - 1)
        def _():
            _zero_agg(rows[0], agg, r0, ZLAST, CHUNK2)

        plsc.subcore_barrier()

        _run_chunks(h2, sidx, didx, rows, gsems, ssem, agg, NCHUNK2)

        plsc.subcore_barrier()

        def copy_out(out_hbm):
            @pl.when(s < NS - 1)
            def _():
                pltpu.sync_copy(agg.at[pl.ds(r0, ZB)], out_hbm.at[pl.ds(r0, ZB)])

            @pl.when(s == NS - 1)
            def _():
                pltpu.sync_copy(agg.at[pl.ds(r0, ZLAST)],
                                out_hbm.at[pl.ds(r0, ZLAST)])

        @pl.when(c == 0)
        def _():
            copy_out(out_a)

        @pl.when(c == 1)
        def _():
            copy_out(out_b)

    return k


def kernel(x, edge_index, W1, b1, W2, b2):
    ei = edge_index.astype(jnp.int32)
    src1 = ei[0].reshape(NS, NCHUNK1, CHUNK1)
    dst1 = ei[1].reshape(NS, NCHUNK1, CHUNK1)
    src2 = ei[0].reshape(NC, NS, NCHUNK2, CHUNK2)
    dst2 = ei[1].reshape(NC, NS, NCHUNK2, CHUNK2)

    lo, hi = _linear1(x, W1, b1.reshape(1, HIDDEN))
    a_lo, a_hi = _scatter1()(lo, hi, src1, dst1)
    h2 = _linear2(a_lo, a_hi, W2[: HIDDEN // 2], W2[HIDDEN // 2:],
                  b2.reshape(1, NUM_CLASSES))
    pa, pb = _scatter2()(h2, src2, dst2)
    return _final_add(pa, pb)
